# baseline (speedup 1.0000x reference)
import functools

import jax
import jax.numpy as jnp
from jax import lax
from jax.experimental import pallas as pl
from jax.experimental.pallas import tpu as pltpu
from jax.experimental.pallas import tpu_sc as plsc

B = 2
N = 32768
C = 512
DHW = 32
BN = B * N

NW = 32
PPW = BN // NW
P = 16
NBLK = PPW // P

_ROWS = (512, 128)


def _prep_body(x_ref, y_ref, z_ref, idx_ref, w_ref):
    x = x_ref[...]
    y = y_ref[...]
    z = z_ref[...]
    scale = 0.5 * (DHW - 1)

    def split(v):
        iv = (v + 1.0) * scale
        v0f = jnp.floor(iv)
        f1 = iv - v0f
        f0 = 1.0 - f1
        v0 = jnp.clip(v0f.astype(jnp.int32), 0, DHW - 1)
        v1 = jnp.clip(v0 + 1, 0, DHW - 1)
        return (v0, v1), (f0, f1)

    (x0, x1), (fx0, fx1) = split(x)
    (y0, y1), (fy0, fy1) = split(y)
    (z0, z1), (fz0, fz1) = split(z)

    r = lax.broadcasted_iota(jnp.int32, _ROWS, 0)
    boff = jnp.where(r >= _ROWS[0] // B, N, 0)

    xs = (x0, x1)
    ys = (y0, y1)
    zs = (z0, z1)
    fxs = (fx0, fx1)
    fys = (fy0, fy1)
    fzs = (fz0, fz1)
    for zb in range(2):
        for yb in range(2):
            for xb in range(2):
                k = zb * 4 + yb * 2 + xb
                idx_ref[k] = boff + zs[zb] * (DHW * DHW) + ys[yb] * DHW + xs[xb]
                w_ref[k] = fzs[zb] * fys[yb] * fxs[xb]


def _prep(x, y, z):
    return pl.pallas_call(
        _prep_body,
        out_shape=(
            jax.ShapeDtypeStruct((8,) + _ROWS, jnp.int32),
            jax.ShapeDtypeStruct((8,) + _ROWS, jnp.float32),
        ),
    )(x, y, z)


def _sc_body(idx_hbm, w_hbm, table_hbm, out_hbm, idx_v, w_v, rows_v, out_v, sem):
    wid = lax.axis_index("s") * 2 + lax.axis_index("c")
    base0 = wid * PPW
    pltpu.sync_copy(idx_hbm.at[wid], idx_v)
    pltpu.sync_copy(w_hbm.at[wid], w_v)

    def blk(g, carry):
        base = base0 + g * P
        pltpu.async_copy(table_hbm.at[idx_v.at[g]], rows_v, sem).wait()

        wrows = [w_v[g, pl.ds(k * P, P)] for k in range(8)]

        def pbody(p, carry2):
            ps = jnp.full((16,), p, jnp.int32)
            wv = [wr.at[ps].get(mode="promise_in_bounds") for wr in wrows]

            def jbody(j, carry3):
                s = pl.ds(j * 16, 16)
                acc = rows_v[p, s] * wv[0]
                for k in range(1, 8):
                    acc = acc + rows_v[k * P + p, s] * wv[k]
                out_v[p, s] = acc
                return carry3

            return lax.fori_loop(0, C // 16, jbody, carry2)

        carry = lax.fori_loop(0, P, pbody, carry)
        pltpu.sync_copy(out_v, out_hbm.at[pl.ds(base, P)])
        return carry

    lax.fori_loop(0, NBLK, blk, 0)


_sc_gather = functools.partial(
    pl.kernel,
    out_type=jax.ShapeDtypeStruct((BN, C), jnp.float32),
    mesh=plsc.VectorSubcoreMesh(core_axis_name="c", subcore_axis_name="s"),
    scratch_types=[
        pltpu.VMEM((NBLK, 8 * P), jnp.int32),
        pltpu.VMEM((NBLK, 8 * P), jnp.float32),
        pltpu.VMEM((8 * P, C), jnp.float32),
        pltpu.VMEM((P, C), jnp.float32),
        pltpu.SemaphoreType.DMA,
    ],
)(_sc_body)


def kernel(voxel_features, vertices):
    table = jnp.transpose(voxel_features, (0, 2, 3, 4, 1)).reshape(BN, C)
    v = vertices.reshape(BN, 3)
    x = v[:, 0].reshape(_ROWS)
    y = v[:, 1].reshape(_ROWS)
    z = v[:, 2].reshape(_ROWS)
    idx8, w8 = _prep(x, y, z)
    idx_t = idx8.reshape(8, NW, NBLK, P).transpose(1, 2, 0, 3).reshape(NW, NBLK, 8 * P)
    w_t = w8.reshape(8, NW, NBLK, P).transpose(1, 2, 0, 3).reshape(NW, NBLK, 8 * P)
    out = _sc_gather(idx_t, w_t, table)
    return out.reshape(B, N, C)


# speedup vs baseline: 1.4283x; 1.4283x over previous
import functools

import jax
import jax.numpy as jnp
from jax import lax
from jax.experimental import pallas as pl
from jax.experimental.pallas import tpu as pltpu
from jax.experimental.pallas import tpu_sc as plsc

B = 2
N = 32768
C = 512
CH = C // 2
DHW = 32
BN = B * N

NW = 32
PPW = BN // NW
P = 16
NBLK = PPW // P
NJOBS = 2 * NBLK

_ROWS = (512, 128)


def _prep_body(x_ref, y_ref, z_ref, idx_ref, w_ref):
    x = x_ref[...]
    y = y_ref[...]
    z = z_ref[...]
    scale = 0.5 * (DHW - 1)

    def split(v):
        iv = (v + 1.0) * scale
        v0f = jnp.floor(iv)
        f1 = iv - v0f
        f0 = 1.0 - f1
        v0 = jnp.clip(v0f.astype(jnp.int32), 0, DHW - 1)
        v1 = jnp.clip(v0 + 1, 0, DHW - 1)
        return (v0, v1), (f0, f1)

    (x0, x1), (fx0, fx1) = split(x)
    (y0, y1), (fy0, fy1) = split(y)
    (z0, z1), (fz0, fz1) = split(z)

    r = lax.broadcasted_iota(jnp.int32, _ROWS, 0)
    boff = jnp.where(r >= _ROWS[0] // B, N, 0)

    xs = (x0, x1)
    ys = (y0, y1)
    zs = (z0, z1)
    fxs = (fx0, fx1)
    fys = (fy0, fy1)
    fzs = (fz0, fz1)
    for zb in range(2):
        for yb in range(2):
            for xb in range(2):
                k = zb * 4 + yb * 2 + xb
                row = boff + zs[zb] * (DHW * DHW) + ys[yb] * DHW + xs[xb]
                idx_ref[0, k] = row * 2
                idx_ref[1, k] = row * 2 + 1
                w_ref[k] = fzs[zb] * fys[yb] * fxs[xb]


def _prep(x, y, z):
    return pl.pallas_call(
        _prep_body,
        out_shape=(
            jax.ShapeDtypeStruct((2, 8) + _ROWS, jnp.int32),
            jax.ShapeDtypeStruct((8,) + _ROWS, jnp.float32),
        ),
    )(x, y, z)


def _sc_body(idx_hbm, w_hbm, table_hbm, out_hbm, idx_v, w_v, rows_v, out_v,
             gsem, osem):
    wid = lax.axis_index("s") * 2 + lax.axis_index("c")
    base0 = wid * PPW
    pltpu.sync_copy(idx_hbm.at[wid], idx_v)
    pltpu.sync_copy(w_hbm.at[wid], w_v)

    def gather_desc(t):
        g = t // 2
        h = t % 2
        return pltpu.make_async_copy(
            table_hbm.at[idx_v.at[g, h]], rows_v.at[h], gsem.at[h])

    def out_desc(t):
        g = t // 2
        h = t % 2
        dst = out_hbm.at[pl.ds(base0 + g * P, P), pl.ds(h * CH, CH)]
        return pltpu.make_async_copy(out_v.at[h], dst, osem.at[h])

    gather_desc(0).start()

    def job(t, carry):
        h = t % 2
        g = t // 2

        @pl.when(t + 1 < NJOBS)
        def _():
            gather_desc(t + 1).start()

        gather_desc(t).wait()

        @pl.when(t >= 2)
        def _():
            out_desc(t - 2).wait()

        wrows = [w_v[g, pl.ds(k * P, P)] for k in range(8)]

        @plsc.parallel_loop(0, P, unroll=2)
        def pbody(p):
            ps = jnp.full((16,), p, jnp.int32)
            wv = [wr.at[ps].get(mode="promise_in_bounds") for wr in wrows]
            for j in range(CH // 16):
                s = pl.ds(j * 16, 16)
                acc = rows_v[h, p, s] * wv[0]
                for k in range(1, 8):
                    acc = acc + rows_v[h, k * P + p, s] * wv[k]
                out_v[h, p, s] = acc

        out_desc(t).start()
        return carry

    lax.fori_loop(0, NJOBS, job, 0)
    out_desc(NJOBS - 2).wait()
    out_desc(NJOBS - 1).wait()


_sc_gather = functools.partial(
    pl.kernel,
    out_type=jax.ShapeDtypeStruct((BN, C), jnp.float32),
    mesh=plsc.VectorSubcoreMesh(core_axis_name="c", subcore_axis_name="s"),
    scratch_types=[
        pltpu.VMEM((NBLK, 2, 8 * P), jnp.int32),
        pltpu.VMEM((NBLK, 8 * P), jnp.float32),
        pltpu.VMEM((2, 8 * P, CH), jnp.float32),
        pltpu.VMEM((2, P, CH), jnp.float32),
        pltpu.SemaphoreType.DMA((2,)),
        pltpu.SemaphoreType.DMA((2,)),
    ],
)(_sc_body)


def kernel(voxel_features, vertices):
    table = jnp.transpose(voxel_features, (0, 2, 3, 4, 1)).reshape(2 * BN, CH)
    v = vertices.reshape(BN, 3)
    x = v[:, 0].reshape(_ROWS)
    y = v[:, 1].reshape(_ROWS)
    z = v[:, 2].reshape(_ROWS)
    idx16, w8 = _prep(x, y, z)
    idx_t = (idx16.reshape(2, 8, NW, NBLK, P)
             .transpose(2, 3, 0, 1, 4).reshape(NW, NBLK, 2, 8 * P))
    w_t = w8.reshape(8, NW, NBLK, P).transpose(1, 2, 0, 3).reshape(NW, NBLK, 8 * P)
    out = _sc_gather(idx_t, w_t, table)
    return out.reshape(B, N, C)
